# baseline (speedup 1.0000x reference)
import functools

import jax
import jax.numpy as jnp
from jax import lax
from jax.experimental import pallas as pl
from jax.experimental.pallas import tpu as pltpu
from jax.experimental.pallas import tpu_sc as plsc

N_NODES = 10000
N_PAIRS = 320000
F = 128


_PRE_BLK = 1000


def _premul_body(x_ref, w1a_ref, w1b_ref, xa_ref, xb_ref):
    x = x_ref[...]
    xa_ref[...] = jnp.dot(x, w1a_ref[...], preferred_element_type=jnp.float32)
    xb_ref[...] = jnp.dot(x, w1b_ref[...], preferred_element_type=jnp.float32)


def _premul(x, w1a, w1b):
    grid = (N_NODES // _PRE_BLK,)
    return pl.pallas_call(
        _premul_body,
        grid=grid,
        in_specs=[
            pl.BlockSpec((_PRE_BLK, F), lambda i: (i, 0)),
            pl.BlockSpec((F, F), lambda i: (0, 0)),
            pl.BlockSpec((F, F), lambda i: (0, 0)),
        ],
        out_specs=[
            pl.BlockSpec((_PRE_BLK, F), lambda i: (i, 0)),
            pl.BlockSpec((_PRE_BLK, F), lambda i: (i, 0)),
        ],
        out_shape=[
            jax.ShapeDtypeStruct((N_NODES, F), jnp.float32),
            jax.ShapeDtypeStruct((N_NODES, F), jnp.float32),
        ],
    )(x, w1a, w1b)


_CHUNK = 128
_NW = 32
_NP_HALF = N_PAIRS // 2
_PW = _NP_HALF // _NW
_NFULL = _PW // _CHUNK
_NTAIL = _PW % _CHUNK


def _gather_add_body(xa_hbm, xb_hbm, idxi_hbm, idxj_hbm, g_hbm,
                     idxi_v, idxj_v,
                     bufa0, bufb0, bufc0, bufa1, bufb1, bufc1,
                     bufta, buftb,
                     sa0, sb0, sc0, sa1, sb1, sc1):
    wid = lax.axis_index("s") * 2 + lax.axis_index("c")
    wbase = wid * _PW

    pltpu.sync_copy(idxi_hbm.at[pl.ds(wbase, _PW)], idxi_v)
    pltpu.sync_copy(idxj_hbm.at[pl.ds(wbase, _PW)], idxj_v)

    def fire(t, bufa, bufb, sa, sb):
        off = t * _CHUNK
        pltpu.async_copy(xa_hbm.at[idxi_v.at[pl.ds(off, _CHUNK)]], bufa, sa)
        pltpu.async_copy(xb_hbm.at[idxj_v.at[pl.ds(off, _CHUNK)]], bufb, sb)

    def drain_gather(bufa, bufb, sa, sb):
        pltpu.make_async_copy(xa_hbm.at[pl.ds(0, _CHUNK)], bufa, sa).wait()
        pltpu.make_async_copy(xa_hbm.at[pl.ds(0, _CHUNK)], bufb, sb).wait()

    def add_rows(nrows, bufa, bufb, bufc):
        def add_row(r, c2):
            for k in range(F // 16):
                sl = pl.ds(k * 16, 16)
                bufc[r, sl] = bufa[r, sl] + bufb[r, sl]
            return c2

        lax.fori_loop(0, nrows, add_row, 0)

    toff = _NFULL * _CHUNK
    cpa = pltpu.async_copy(xa_hbm.at[idxi_v.at[pl.ds(toff, _NTAIL)]], bufta, sa0)
    cpb = pltpu.async_copy(xb_hbm.at[idxj_v.at[pl.ds(toff, _NTAIL)]], buftb, sb0)
    cpa.wait()
    cpb.wait()
    add_rows(_NTAIL, bufta, buftb, bufta)
    pltpu.sync_copy(bufta, g_hbm.at[pl.ds(wbase + toff, _NTAIL)])

    fire(0, bufa0, bufb0, sa0, sb0)
    fire(1, bufa1, bufb1, sa1, sb1)

    def do_set(u, t, bufa, bufb, bufc, sa, sb, sc):
        drain_gather(bufa, bufb, sa, sb)

        @pl.when(u > 0)
        def _wait_prev_store():
            pltpu.make_async_copy(bufc, g_hbm.at[pl.ds(0, _CHUNK)], sc).wait()

        add_rows(_CHUNK, bufa, bufb, bufc)

        @pl.when(t + 2 < _NFULL)
        def _fire_next():
            fire(t + 2, bufa, bufb, sa, sb)

        pltpu.async_copy(bufc, g_hbm.at[pl.ds(wbase + t * _CHUNK, _CHUNK)], sc)

    def body(u, carry):
        do_set(u, 2 * u, bufa0, bufb0, bufc0, sa0, sb0, sc0)
        do_set(u, 2 * u + 1, bufa1, bufb1, bufc1, sa1, sb1, sc1)
        return carry

    lax.fori_loop(0, _NFULL // 2, body, 0)

    if _NFULL % 2 == 1:
        do_set(_NFULL // 2, _NFULL - 1, bufa0, bufb0, bufc0, sa0, sb0, sc0)

    pltpu.make_async_copy(bufc0, g_hbm.at[pl.ds(0, _CHUNK)], sc0).wait()
    pltpu.make_async_copy(bufc1, g_hbm.at[pl.ds(0, _CHUNK)], sc1).wait()


@functools.cache
def _make_gather_add():
    return functools.partial(
        pl.kernel,
        out_type=jax.ShapeDtypeStruct((_NP_HALF, F), jnp.float32),
        mesh=plsc.VectorSubcoreMesh(core_axis_name="c", subcore_axis_name="s"),
        compiler_params=pltpu.CompilerParams(needs_layout_passes=False),
        scratch_types=[
            pltpu.VMEM((_PW,), jnp.int32),
            pltpu.VMEM((_PW,), jnp.int32),
            pltpu.VMEM((_CHUNK, F), jnp.float32),
            pltpu.VMEM((_CHUNK, F), jnp.float32),
            pltpu.VMEM((_CHUNK, F), jnp.float32),
            pltpu.VMEM((_CHUNK, F), jnp.float32),
            pltpu.VMEM((_CHUNK, F), jnp.float32),
            pltpu.VMEM((_CHUNK, F), jnp.float32),
            pltpu.VMEM((_NTAIL, F), jnp.float32),
            pltpu.VMEM((_NTAIL, F), jnp.float32),
            pltpu.SemaphoreType.DMA,
            pltpu.SemaphoreType.DMA,
            pltpu.SemaphoreType.DMA,
            pltpu.SemaphoreType.DMA,
            pltpu.SemaphoreType.DMA,
            pltpu.SemaphoreType.DMA,
        ],
    )(_gather_add_body)


_CAST_BLK = 4000


def _cast_body(x_ref, o_ref):
    o_ref[...] = x_ref[...].astype(jnp.bfloat16)


def _cast_bf16(w_ij):
    return pl.pallas_call(
        _cast_body,
        grid=(N_PAIRS // _CAST_BLK,),
        in_specs=[pl.BlockSpec((_CAST_BLK, F), lambda i: (i, 0))],
        out_specs=pl.BlockSpec((_CAST_BLK, F), lambda i: (i, 0)),
        out_shape=jax.ShapeDtypeStruct((N_PAIRS, F), jnp.bfloat16),
    )(w_ij)


_MLP_BLK = 2000
_MLP_NB = _NP_HALF // _MLP_BLK


def _mlp_body(w_ref, g_ref, w1c_ref, b1_ref, w2_ref, b2_ref, out_ref):
    c = jnp.dot(w_ref[...], w1c_ref[...].astype(jnp.bfloat16),
                preferred_element_type=jnp.float32)
    y = c + g_ref[...] + b1_ref[...]
    h = y * jax.nn.sigmoid(y)
    s = lax.dot_general(w2_ref[...].astype(jnp.bfloat16), h.astype(jnp.bfloat16),
                        dimension_numbers=(((1,), (1,)), ((), ())),
                        preferred_element_type=jnp.float32)
    out_ref[0, 0, :] = s[0] + b2_ref[0]


def _mlp(w_ij, g_half, w1c, b1_row, w2_row, b2, half):
    off = half * _MLP_NB
    out = pl.pallas_call(
        _mlp_body,
        grid=(_MLP_NB,),
        in_specs=[
            pl.BlockSpec((_MLP_BLK, F), lambda i: (i + off, 0)),
            pl.BlockSpec((_MLP_BLK, F), lambda i: (i, 0)),
            pl.BlockSpec((F, F), lambda i: (0, 0)),
            pl.BlockSpec((1, F), lambda i: (0, 0)),
            pl.BlockSpec((1, F), lambda i: (0, 0)),
            pl.BlockSpec(memory_space=pltpu.SMEM),
        ],
        out_specs=pl.BlockSpec((1, 1, _MLP_BLK), lambda i: (i, 0, 0)),
        out_shape=jax.ShapeDtypeStruct((_MLP_NB, 1, _MLP_BLK), jnp.float32),
    )(w_ij, g_half, w1c, b1_row, w2_row, b2)
    return out.reshape(_NP_HALF)


def kernel(x, w_ij, idx_i, idx_j, W1, b1, W2, b2):
    w1a = W1[:F]
    w1b = W1[F:2 * F]
    w1c = W1[2 * F:]
    xa, xb = _premul(x, w1a, w1b)
    idx_i = idx_i.astype(jnp.int32)
    idx_j = idx_j.astype(jnp.int32)
    sc = _make_gather_add()
    b1_row = b1.reshape(1, F)
    w2_row = W2.reshape(1, F)
    w16 = _cast_bf16(w_ij)
    g0 = sc(xa, xb, idx_i[:_NP_HALF], idx_j[:_NP_HALF])
    g1 = sc(xa, xb, idx_i[_NP_HALF:], idx_j[_NP_HALF:])
    o0 = _mlp(w16, g0, w1c, b1_row, w2_row, b2, 0)
    o1 = _mlp(w16, g1, w1c, b1_row, w2_row, b2, 1)
    return jnp.concatenate([o0, o1])


# speedup vs baseline: 1.1672x; 1.1672x over previous
import functools

import jax
import jax.numpy as jnp
from jax import lax
from jax.experimental import pallas as pl
from jax.experimental.pallas import tpu as pltpu
from jax.experimental.pallas import tpu_sc as plsc

N_NODES = 10000
N_PAIRS = 320000
F = 128


_PRE_BLK = 1000


def _premul_body(x_ref, w1a_ref, w1b_ref, xa_ref, xb_ref):
    x = x_ref[...]
    xa_ref[...] = jnp.dot(x, w1a_ref[...], preferred_element_type=jnp.float32)
    xb_ref[...] = jnp.dot(x, w1b_ref[...], preferred_element_type=jnp.float32)


def _premul(x, w1a, w1b):
    grid = (N_NODES // _PRE_BLK,)
    return pl.pallas_call(
        _premul_body,
        grid=grid,
        in_specs=[
            pl.BlockSpec((_PRE_BLK, F), lambda i: (i, 0)),
            pl.BlockSpec((F, F), lambda i: (0, 0)),
            pl.BlockSpec((F, F), lambda i: (0, 0)),
        ],
        out_specs=[
            pl.BlockSpec((_PRE_BLK, F), lambda i: (i, 0)),
            pl.BlockSpec((_PRE_BLK, F), lambda i: (i, 0)),
        ],
        out_shape=[
            jax.ShapeDtypeStruct((N_NODES, F), jnp.float32),
            jax.ShapeDtypeStruct((N_NODES, F), jnp.float32),
        ],
    )(x, w1a, w1b)


_CHUNK = 128
_NW = 32
_NP_SC = N_PAIRS
_PW = _NP_SC // _NW
_NFULL = _PW // _CHUNK
_NTAIL = _PW % _CHUNK


def _gather_add_body(xa_hbm, xb_hbm, idxi_hbm, idxj_hbm, g_hbm,
                     idxi_v, idxj_v,
                     bufa0, bufb0, bufc0, bufa1, bufb1, bufc1,
                     bufta, buftb,
                     sa0, sb0, sc0, sa1, sb1, sc1):
    wid = lax.axis_index("s") * 2 + lax.axis_index("c")
    wbase = wid * _PW

    pltpu.sync_copy(idxi_hbm.at[pl.ds(wbase, _PW)], idxi_v)
    pltpu.sync_copy(idxj_hbm.at[pl.ds(wbase, _PW)], idxj_v)

    def fire(t, bufa, bufb, sa, sb):
        off = t * _CHUNK
        pltpu.async_copy(xa_hbm.at[idxi_v.at[pl.ds(off, _CHUNK)]], bufa, sa)
        pltpu.async_copy(xb_hbm.at[idxj_v.at[pl.ds(off, _CHUNK)]], bufb, sb)

    def drain_gather(bufa, bufb, sa, sb):
        pltpu.make_async_copy(xa_hbm.at[pl.ds(0, _CHUNK)], bufa, sa).wait()
        pltpu.make_async_copy(xa_hbm.at[pl.ds(0, _CHUNK)], bufb, sb).wait()

    def add_rows(nrows, bufa, bufb, bufc):
        def add_row(r, c2):
            for k in range(F // 16):
                sl = pl.ds(k * 16, 16)
                bufc[r, sl] = bufa[r, sl] + bufb[r, sl]
            return c2

        lax.fori_loop(0, nrows, add_row, 0)

    toff = _NFULL * _CHUNK
    cpa = pltpu.async_copy(xa_hbm.at[idxi_v.at[pl.ds(toff, _NTAIL)]], bufta, sa0)
    cpb = pltpu.async_copy(xb_hbm.at[idxj_v.at[pl.ds(toff, _NTAIL)]], buftb, sb0)
    cpa.wait()
    cpb.wait()
    add_rows(_NTAIL, bufta, buftb, bufta)
    pltpu.sync_copy(bufta, g_hbm.at[pl.ds(wbase + toff, _NTAIL)])

    fire(0, bufa0, bufb0, sa0, sb0)
    fire(1, bufa1, bufb1, sa1, sb1)

    def do_set(u, t, bufa, bufb, bufc, sa, sb, sc):
        drain_gather(bufa, bufb, sa, sb)

        @pl.when(u > 0)
        def _wait_prev_store():
            pltpu.make_async_copy(bufc, g_hbm.at[pl.ds(0, _CHUNK)], sc).wait()

        add_rows(_CHUNK, bufa, bufb, bufc)

        @pl.when(t + 2 < _NFULL)
        def _fire_next():
            fire(t + 2, bufa, bufb, sa, sb)

        pltpu.async_copy(bufc, g_hbm.at[pl.ds(wbase + t * _CHUNK, _CHUNK)], sc)

    def body(u, carry):
        do_set(u, 2 * u, bufa0, bufb0, bufc0, sa0, sb0, sc0)
        do_set(u, 2 * u + 1, bufa1, bufb1, bufc1, sa1, sb1, sc1)
        return carry

    lax.fori_loop(0, _NFULL // 2, body, 0)

    if _NFULL % 2 == 1:
        do_set(_NFULL // 2, _NFULL - 1, bufa0, bufb0, bufc0, sa0, sb0, sc0)

    pltpu.make_async_copy(bufc0, g_hbm.at[pl.ds(0, _CHUNK)], sc0).wait()
    pltpu.make_async_copy(bufc1, g_hbm.at[pl.ds(0, _CHUNK)], sc1).wait()


@functools.cache
def _make_gather_add():
    return functools.partial(
        pl.kernel,
        out_type=jax.ShapeDtypeStruct((_NP_SC, F), jnp.float32),
        mesh=plsc.VectorSubcoreMesh(core_axis_name="c", subcore_axis_name="s"),
        compiler_params=pltpu.CompilerParams(needs_layout_passes=False),
        scratch_types=[
            pltpu.VMEM((_PW,), jnp.int32),
            pltpu.VMEM((_PW,), jnp.int32),
            pltpu.VMEM((_CHUNK, F), jnp.float32),
            pltpu.VMEM((_CHUNK, F), jnp.float32),
            pltpu.VMEM((_CHUNK, F), jnp.float32),
            pltpu.VMEM((_CHUNK, F), jnp.float32),
            pltpu.VMEM((_CHUNK, F), jnp.float32),
            pltpu.VMEM((_CHUNK, F), jnp.float32),
            pltpu.VMEM((_NTAIL, F), jnp.float32),
            pltpu.VMEM((_NTAIL, F), jnp.float32),
            pltpu.SemaphoreType.DMA,
            pltpu.SemaphoreType.DMA,
            pltpu.SemaphoreType.DMA,
            pltpu.SemaphoreType.DMA,
            pltpu.SemaphoreType.DMA,
            pltpu.SemaphoreType.DMA,
        ],
    )(_gather_add_body)


_MLP_BLK = 2560
_MLP_NB = N_PAIRS // _MLP_BLK


def _mlp_body(w_ref, g_ref, w1c_ref, b1_ref, w2_ref, b2_ref, out_ref):
    c = jnp.dot(w_ref[...].astype(jnp.bfloat16), w1c_ref[...].astype(jnp.bfloat16),
                preferred_element_type=jnp.float32)
    y = c + g_ref[...] + b1_ref[...]
    h = y * jax.nn.sigmoid(y)
    s = lax.dot_general(w2_ref[...].astype(jnp.bfloat16), h.astype(jnp.bfloat16),
                        dimension_numbers=(((1,), (1,)), ((), ())),
                        preferred_element_type=jnp.float32)
    out_ref[0, 0, :] = s[0] + b2_ref[0]


def _mlp(w_ij, g, w1c, b1_row, w2_row, b2):
    out = pl.pallas_call(
        _mlp_body,
        grid=(_MLP_NB,),
        in_specs=[
            pl.BlockSpec((_MLP_BLK, F), lambda i: (i, 0)),
            pl.BlockSpec((_MLP_BLK, F), lambda i: (i, 0)),
            pl.BlockSpec((F, F), lambda i: (0, 0)),
            pl.BlockSpec((1, F), lambda i: (0, 0)),
            pl.BlockSpec((1, F), lambda i: (0, 0)),
            pl.BlockSpec(memory_space=pltpu.SMEM),
        ],
        out_specs=pl.BlockSpec((1, 1, _MLP_BLK), lambda i: (i, 0, 0)),
        out_shape=jax.ShapeDtypeStruct((_MLP_NB, 1, _MLP_BLK), jnp.float32),
    )(w_ij, g, w1c, b1_row, w2_row, b2)
    return out.reshape(N_PAIRS)


def kernel(x, w_ij, idx_i, idx_j, W1, b1, W2, b2):
    w1a = W1[:F]
    w1b = W1[F:2 * F]
    w1c = W1[2 * F:]
    xa, xb = _premul(x, w1a, w1b)
    g = _make_gather_add()(xa, xb,
                           idx_i.astype(jnp.int32), idx_j.astype(jnp.int32))
    return _mlp(w_ij, g, w1c, b1.reshape(1, F), W2.reshape(1, F), b2)
